# Initial kernel scaffold; baseline (speedup 1.0000x reference)
import jax
import jax.numpy as jnp
from jax.experimental import pallas as pl


def kernel(q, r):
    raise NotImplementedError("write your pallas kernel here")


# speedup vs baseline: 28.0742x; 28.0742x over previous
import jax
import jax.numpy as jnp
from jax.experimental import pallas as pl
from jax.experimental.pallas import tpu as pltpu

_K = 16
_QB = 64
_BIG_IDX = 2**30


def _knn_block(q_ref, r_ref, idx_ref, d_ref):
    q = q_ref[0]
    r = r_ref[0]
    qn = jnp.sum(q * q, axis=1)
    rn = jnp.sum(r * r, axis=1)
    dot = jax.lax.dot_general(
        q, r, (((1,), (1,)), ((), ())),
        preferred_element_type=jnp.float32,
        precision=jax.lax.Precision.HIGHEST,
    )
    d2 = qn[:, None] + rn[None, :] - 2.0 * dot
    key = jnp.sqrt(jnp.maximum(d2, 0.0))
    lane = jax.lax.broadcasted_iota(jnp.int32, key.shape, 1)
    idx_cols = []
    d_cols = []
    for _ in range(_K):
        m = jnp.min(key, axis=1, keepdims=True)
        sel = jnp.where(key == m, lane, _BIG_IDX)
        j = jnp.min(sel, axis=1, keepdims=True)
        idx_cols.append(j)
        d_cols.append(m)
        key = jnp.where(lane == j, float("inf"), key)
    idx_ref[0] = jnp.concatenate(idx_cols, axis=1)
    d_ref[0] = jnp.concatenate(d_cols, axis=1)


def kernel(q, r):
    B, Q, D = q.shape
    R = r.shape[1]
    grid = (B, Q // _QB)
    idx, d = pl.pallas_call(
        _knn_block,
        grid=grid,
        in_specs=[
            pl.BlockSpec((1, _QB, D), lambda b, i: (b, i, 0)),
            pl.BlockSpec((1, R, D), lambda b, i: (b, 0, 0)),
        ],
        out_specs=[
            pl.BlockSpec((1, _QB, _K), lambda b, i: (b, i, 0)),
            pl.BlockSpec((1, _QB, _K), lambda b, i: (b, i, 0)),
        ],
        out_shape=[
            jax.ShapeDtypeStruct((B, Q, _K), jnp.int32),
            jax.ShapeDtypeStruct((B, Q, _K), jnp.float32),
        ],
    )(q, r)
    return idx, d


# speedup vs baseline: 29.4101x; 1.0476x over previous
import jax
import jax.numpy as jnp
from jax.experimental import pallas as pl
from jax.experimental.pallas import tpu as pltpu

_K = 16
_QB = 64
_LN = 128
_C = 4
_BIG_IDX = 2**30
_PAD = 72


def _knn_block(q_ref, r_ref, idx_ref, d_ref, rp_ref):
    R = r_ref.shape[1]
    CH = R // _LN
    i = pl.program_id(1)

    @pl.when(i == 0)
    def _build_rp():
        r = r_ref[0]
        rn = jnp.sum(r * r, axis=1, keepdims=True)
        ones = jnp.ones((R, 1), jnp.float32)
        pad = jnp.zeros((R, _PAD - 66), jnp.float32)
        rp_ref[...] = jnp.concatenate([r, ones, rn, pad], axis=1)

    q = q_ref[0]
    qn = jnp.sum(q * q, axis=1, keepdims=True)
    ones = jnp.ones((q.shape[0], 1), jnp.float32)
    padq = jnp.zeros((q.shape[0], _PAD - 66), jnp.float32)
    qp = jnp.concatenate([-2.0 * q, qn, ones, padq], axis=1)
    d2 = jax.lax.dot_general(
        qp, rp_ref[...], (((1,), (1,)), ((), ())),
        preferred_element_type=jnp.float32,
        precision=jax.lax.Precision.HIGHEST,
    )

    d3 = d2.reshape(_QB, CH, _LN)
    lane3 = jax.lax.broadcasted_iota(jnp.int32, (_QB, CH, _LN), 2)
    work = d3
    cand_v = []
    cand_l = []
    for _ in range(_C):
        m = jnp.min(work, axis=2)
        sel = jnp.where(work == m[:, :, None], lane3, _BIG_IDX)
        l = jnp.min(sel, axis=2)
        cand_v.append(m)
        cand_l.append(l)
        work = jnp.where(lane3 == l[:, :, None], float("inf"), work)

    V = jnp.stack(cand_v, axis=1)
    L = jnp.stack(cand_l, axis=1)
    chunk3 = jax.lax.broadcasted_iota(jnp.int32, (_QB, _C, CH), 2)
    G = chunk3 * _LN + L
    V = V.reshape(_QB, _C * CH)
    G = G.reshape(_QB, _C * CH)

    idx_cols = []
    d_cols = []
    for _ in range(_K):
        m = jnp.min(V, axis=1, keepdims=True)
        sel = jnp.where(V == m, G, _BIG_IDX)
        g = jnp.min(sel, axis=1, keepdims=True)
        idx_cols.append(g)
        d_cols.append(m)
        V = jnp.where(G == g, float("inf"), V)
    idx16 = jnp.concatenate(idx_cols, axis=1)
    d16 = jnp.concatenate(d_cols, axis=1)

    idx_ref[0] = idx16
    d_ref[0] = jnp.sqrt(jnp.maximum(d16, 0.0))

    seg_iota = jax.lax.broadcasted_iota(jnp.int32, (_QB, CH), 1)
    cnt = jnp.zeros((_QB, CH), jnp.int32)
    for j in range(_K):
        cnt = cnt + (idx16[:, j:j + 1] // _LN == seg_iota).astype(jnp.int32)
    overflow = jnp.any(cnt >= _C)

    @pl.when(overflow)
    def _exact_fallback():
        lane2 = jax.lax.broadcasted_iota(jnp.int32, d2.shape, 1)
        work2 = d2
        icols = []
        dcols = []
        for _ in range(_K):
            m = jnp.min(work2, axis=1, keepdims=True)
            sel = jnp.where(work2 == m, lane2, _BIG_IDX)
            g = jnp.min(sel, axis=1, keepdims=True)
            icols.append(g)
            dcols.append(m)
            work2 = jnp.where(lane2 == g, float("inf"), work2)
        idx_ref[0] = jnp.concatenate(icols, axis=1)
        d_ref[0] = jnp.sqrt(jnp.maximum(jnp.concatenate(dcols, axis=1), 0.0))


def kernel(q, r):
    B, Q, D = q.shape
    R = r.shape[1]
    grid = (B, Q // _QB)
    idx, d = pl.pallas_call(
        _knn_block,
        grid=grid,
        in_specs=[
            pl.BlockSpec((1, _QB, D), lambda b, i: (b, i, 0)),
            pl.BlockSpec((1, R, D), lambda b, i: (b, 0, 0)),
        ],
        out_specs=[
            pl.BlockSpec((1, _QB, _K), lambda b, i: (b, i, 0)),
            pl.BlockSpec((1, _QB, _K), lambda b, i: (b, i, 0)),
        ],
        out_shape=[
            jax.ShapeDtypeStruct((B, Q, _K), jnp.int32),
            jax.ShapeDtypeStruct((B, Q, _K), jnp.float32),
        ],
        scratch_shapes=[pltpu.VMEM((R, _PAD), jnp.float32)],
    )(q, r)
    return idx, d


# speedup vs baseline: 41.0151x; 1.3946x over previous
import jax
import jax.numpy as jnp
from jax.experimental import pallas as pl
from jax.experimental.pallas import tpu as pltpu

_K = 16
_QB = 64
_LN = 128
_C = 4
_BIG_IDX = 2**30
_PAD = 72


def _knn_block(q_ref, r_ref, idx_ref, d_ref, rp_ref):
    R = r_ref.shape[1]
    CH = R // _LN
    i = pl.program_id(1)

    @pl.when(i == 0)
    def _build_rp():
        r = r_ref[0]
        rn = jnp.sum(r * r, axis=1, keepdims=True)
        ones = jnp.ones((R, 1), jnp.float32)
        pad = jnp.zeros((R, _PAD - 66), jnp.float32)
        rp_ref[...] = jnp.concatenate([r, ones, rn, pad], axis=1)

    q = q_ref[0]
    qn = jnp.sum(q * q, axis=1, keepdims=True)
    ones = jnp.ones((q.shape[0], 1), jnp.float32)
    padq = jnp.zeros((q.shape[0], _PAD - 66), jnp.float32)
    qp = jnp.concatenate([-2.0 * q, qn, ones, padq], axis=1)
    d2 = jax.lax.dot_general(
        qp, rp_ref[...], (((1,), (1,)), ((), ())),
        preferred_element_type=jnp.float32,
        precision=jax.lax.Precision.HIGHEST,
    )

    d3 = d2.reshape(_QB, CH, _LN)
    ch3 = jax.lax.broadcasted_iota(jnp.int32, (_QB, CH, _LN), 1)
    work = d3
    cand_v = []
    cand_c = []
    for _ in range(_C):
        m = jnp.min(work, axis=1)
        sel = jnp.where(work == m[:, None, :], ch3, _BIG_IDX)
        c = jnp.min(sel, axis=1)
        cand_v.append(m)
        cand_c.append(c)
        work = jnp.where(ch3 == c[:, None, :], float("inf"), work)

    V = jnp.stack(cand_v, axis=1)
    Cc = jnp.stack(cand_c, axis=1)
    ln3 = jax.lax.broadcasted_iota(jnp.int32, (_QB, _C, _LN), 2)
    G = Cc * _LN + ln3
    V = V.reshape(_QB, _C * _LN)
    G = G.reshape(_QB, _C * _LN)

    idx_cols = []
    d_cols = []
    for _ in range(_K):
        m = jnp.min(V, axis=1, keepdims=True)
        sel = jnp.where(V == m, G, _BIG_IDX)
        g = jnp.min(sel, axis=1, keepdims=True)
        idx_cols.append(g)
        d_cols.append(m)
        V = jnp.where(G == g, float("inf"), V)
    idx16 = jnp.concatenate(idx_cols, axis=1)
    d16 = jnp.concatenate(d_cols, axis=1)

    idx_ref[0] = idx16
    d_ref[0] = jnp.sqrt(jnp.maximum(d16, 0.0))

    seg_iota = jax.lax.broadcasted_iota(jnp.int32, (_QB, _LN), 1)
    cnt = jnp.zeros((_QB, _LN), jnp.int32)
    for j in range(_K):
        cnt = cnt + (idx16[:, j:j + 1] % _LN == seg_iota).astype(jnp.int32)
    overflow = jnp.any(cnt >= _C)

    @pl.when(overflow)
    def _exact_fallback():
        lane2 = jax.lax.broadcasted_iota(jnp.int32, d2.shape, 1)
        work2 = d2
        icols = []
        dcols = []
        for _ in range(_K):
            m = jnp.min(work2, axis=1, keepdims=True)
            sel = jnp.where(work2 == m, lane2, _BIG_IDX)
            g = jnp.min(sel, axis=1, keepdims=True)
            icols.append(g)
            dcols.append(m)
            work2 = jnp.where(lane2 == g, float("inf"), work2)
        idx_ref[0] = jnp.concatenate(icols, axis=1)
        d_ref[0] = jnp.sqrt(jnp.maximum(jnp.concatenate(dcols, axis=1), 0.0))


def kernel(q, r):
    B, Q, D = q.shape
    R = r.shape[1]
    grid = (B, Q // _QB)
    idx, d = pl.pallas_call(
        _knn_block,
        grid=grid,
        in_specs=[
            pl.BlockSpec((1, _QB, D), lambda b, i: (b, i, 0)),
            pl.BlockSpec((1, R, D), lambda b, i: (b, 0, 0)),
        ],
        out_specs=[
            pl.BlockSpec((1, _QB, _K), lambda b, i: (b, i, 0)),
            pl.BlockSpec((1, _QB, _K), lambda b, i: (b, i, 0)),
        ],
        out_shape=[
            jax.ShapeDtypeStruct((B, Q, _K), jnp.int32),
            jax.ShapeDtypeStruct((B, Q, _K), jnp.float32),
        ],
        scratch_shapes=[pltpu.VMEM((R, _PAD), jnp.float32)],
    )(q, r)
    return idx, d


# speedup vs baseline: 57.9523x; 1.4129x over previous
import jax
import jax.numpy as jnp
from jax.experimental import pallas as pl
from jax.experimental.pallas import tpu as pltpu

_K = 16
_QB = 128
_LN = 128
_C = 4
_NP = 8
_BIG_IDX = 2**30
_PAD = 72


def _capped_build(qp, rp_ref, cap, CH):
    CHP = CH // _NP
    ln3 = jax.lax.broadcasted_iota(jnp.int32, (CHP, _LN, _QB), 1)
    chp3 = jax.lax.broadcasted_iota(jnp.int32, (CHP, cap, _QB), 0)
    Vs = []
    Gs = []
    for p in range(_NP):
        d2p = jax.lax.dot_general(
            rp_ref[pl.ds(p * CHP * _LN, CHP * _LN), :], qp,
            (((1,), (1,)), ((), ())),
            preferred_element_type=jnp.float32,
            precision=jax.lax.Precision.HIGHEST,
        )
        work = d2p.reshape(CHP, _LN, _QB)
        pv = []
        pl_ = []
        for _ in range(cap):
            m = jnp.min(work, axis=1)
            sel = jnp.where(work == m[:, None, :], ln3, _BIG_IDX)
            l = jnp.min(sel, axis=1)
            pv.append(m)
            pl_.append(l)
            work = jnp.where(ln3 == l[:, None, :], float("inf"), work)
        Vp = jnp.stack(pv, axis=1)
        Lp = jnp.stack(pl_, axis=1)
        Gp = (chp3 + p * CHP) * _LN + Lp
        Vs.append(Vp.reshape(CHP * cap, _QB))
        Gs.append(Gp.reshape(CHP * cap, _QB))
    return jnp.concatenate(Vs, axis=0), jnp.concatenate(Gs, axis=0)


def _extract_topk(V, G):
    idx_rows = []
    d_rows = []
    for _ in range(_K):
        m = jnp.min(V, axis=0, keepdims=True)
        sel = jnp.where(V == m, G, _BIG_IDX)
        g = jnp.min(sel, axis=0, keepdims=True)
        idx_rows.append(g)
        d_rows.append(m)
        V = jnp.where(G == g, float("inf"), V)
    return jnp.concatenate(idx_rows, axis=0), jnp.concatenate(d_rows, axis=0)


def _knn_block(q_ref, r_ref, idx_ref, d_ref, rp_ref):
    R = r_ref.shape[1]
    CH = R // _LN
    i = pl.program_id(1)

    @pl.when(i == 0)
    def _build_rp():
        r = r_ref[0]
        rn = jnp.sum(r * r, axis=1, keepdims=True)
        ones = jnp.ones((R, 1), jnp.float32)
        pad = jnp.zeros((R, _PAD - 66), jnp.float32)
        rp_ref[...] = jnp.concatenate([r, ones, rn, pad], axis=1)

    q = q_ref[0]
    qn = jnp.sum(q * q, axis=1, keepdims=True)
    ones = jnp.ones((q.shape[0], 1), jnp.float32)
    padq = jnp.zeros((q.shape[0], _PAD - 66), jnp.float32)
    qp = jnp.concatenate([-2.0 * q, qn, ones, padq], axis=1)

    V, G = _capped_build(qp, rp_ref, _C, CH)
    idx16, d16 = _extract_topk(V, G)

    idx_ref[0] = idx16.T
    d_ref[0] = jnp.sqrt(jnp.maximum(d16, 0.0)).T

    seg_iota = jax.lax.broadcasted_iota(jnp.int32, (CH, _QB), 0)
    cnt = jnp.zeros((CH, _QB), jnp.int32)
    for j in range(_K):
        cnt = cnt + (idx16[j:j + 1, :] // _LN == seg_iota).astype(jnp.int32)
    overflow = jnp.any(cnt >= _C)

    @pl.when(overflow)
    def _exact_fallback():
        V2, G2 = _capped_build(qp, rp_ref, _K, CH)
        i2, d2_ = _extract_topk(V2, G2)
        idx_ref[0] = i2.T
        d_ref[0] = jnp.sqrt(jnp.maximum(d2_, 0.0)).T


def kernel(q, r):
    B, Q, D = q.shape
    R = r.shape[1]
    grid = (B, Q // _QB)
    idx, d = pl.pallas_call(
        _knn_block,
        grid=grid,
        in_specs=[
            pl.BlockSpec((1, _QB, D), lambda b, i: (b, i, 0)),
            pl.BlockSpec((1, R, D), lambda b, i: (b, 0, 0)),
        ],
        out_specs=[
            pl.BlockSpec((1, _QB, _K), lambda b, i: (b, i, 0)),
            pl.BlockSpec((1, _QB, _K), lambda b, i: (b, i, 0)),
        ],
        out_shape=[
            jax.ShapeDtypeStruct((B, Q, _K), jnp.int32),
            jax.ShapeDtypeStruct((B, Q, _K), jnp.float32),
        ],
        scratch_shapes=[pltpu.VMEM((R, _PAD), jnp.float32)],
    )(q, r)
    return idx, d
